# Initial kernel scaffold; baseline (speedup 1.0000x reference)
import functools

import jax
import jax.numpy as jnp
from jax import lax
from jax.experimental import pallas as pl
from jax.experimental.pallas import tpu as pltpu
from jax.experimental.pallas import tpu_sc as plsc

N = 10000
D = 128
E = 320000

NC = 2
NS = 16
CH = 128
G = 80
EPT = G * CH
E_PAD = NC * NS * EPT
ER2D = E_PAD // CH
N_PAD = 10112
ROWS_PT = N_PAD // NS
DEGW = 16


def _sc_agg_body(h_hbm, src2, dst2, ones_in, zrow, zrow16,
                 agg0, agg1, deg0, deg1,
                 ones_v, srcs, dsts, rows, aggs, degs, sem):
    cid = lax.axis_index("c")
    sid = lax.axis_index("s")
    tid = cid * NS + sid
    zb = sid * ROWS_PT

    pltpu.sync_copy(zrow, aggs.at[pl.ds(zb, ROWS_PT)])
    pltpu.sync_copy(zrow16, degs.at[pl.ds(zb, ROWS_PT)])
    pltpu.sync_copy(ones_in, ones_v)

    rb = tid * G
    pltpu.sync_copy(src2.at[pl.ds(rb, G)], srcs)
    pltpu.sync_copy(dst2.at[pl.ds(rb, G)], dsts)

    plsc.subcore_barrier()

    def chunk(g, carry):
        sidx = srcs.at[g]
        didx = dsts.at[g]
        cp = pltpu.async_copy(h_hbm.at[sidx], rows, sem)
        pltpu.sync_copy(ones_v, degs.at[didx], add=True)
        cp.wait()
        pltpu.sync_copy(rows, aggs.at[didx], add=True)
        return carry

    lax.fori_loop(0, G, chunk, 0)

    plsc.subcore_barrier()

    @pl.when(cid == 0)
    def _():
        pltpu.sync_copy(aggs.at[pl.ds(zb, ROWS_PT)], agg0.at[pl.ds(zb, ROWS_PT)])
        pltpu.sync_copy(degs.at[pl.ds(zb, ROWS_PT)], deg0.at[pl.ds(zb, ROWS_PT)])

    @pl.when(cid == 1)
    def _():
        pltpu.sync_copy(aggs.at[pl.ds(zb, ROWS_PT)], agg1.at[pl.ds(zb, ROWS_PT)])
        pltpu.sync_copy(degs.at[pl.ds(zb, ROWS_PT)], deg1.at[pl.ds(zb, ROWS_PT)])


_sc_agg = pl.kernel(
    _sc_agg_body,
    out_type=[
        jax.ShapeDtypeStruct((N_PAD, D), jnp.float32),
        jax.ShapeDtypeStruct((N_PAD, D), jnp.float32),
        jax.ShapeDtypeStruct((N_PAD, DEGW), jnp.float32),
        jax.ShapeDtypeStruct((N_PAD, DEGW), jnp.float32),
    ],
    mesh=plsc.VectorSubcoreMesh(core_axis_name="c", subcore_axis_name="s"),
    scratch_types=[
        pltpu.VMEM((CH, DEGW), jnp.float32),
        pltpu.VMEM((G, CH), jnp.int32),
        pltpu.VMEM((G, CH), jnp.int32),
        pltpu.VMEM((CH, D), jnp.float32),
        pltpu.VMEM_SHARED((N_PAD, D), jnp.float32),
        pltpu.VMEM_SHARED((N_PAD, DEGW), jnp.float32),
        pltpu.SemaphoreType.DMA,
    ],
)


def _dense_body(a0, a1, d0, d1, x, wl, wr, b, o, *, relu):
    deg = d0[:, 0:1] + d1[:, 0:1]
    mean = (a0[...] + a1[...]) / jnp.maximum(deg, 1.0)
    r = (jnp.dot(mean, wl[...], preferred_element_type=jnp.float32)
         + jnp.dot(x[...], wr[...], preferred_element_type=jnp.float32)
         + b[...])
    o[...] = jnp.maximum(r, 0.0) if relu else r


def _make_dense(relu):
    blk = pl.BlockSpec((CH, D), lambda i: (i, 0))
    dblk = pl.BlockSpec((CH, DEGW), lambda i: (i, 0))
    wblk = pl.BlockSpec((D, D), lambda i: (0, 0))
    bblk = pl.BlockSpec((1, D), lambda i: (0, 0))
    return pl.pallas_call(
        functools.partial(_dense_body, relu=relu),
        grid=(N_PAD // CH,),
        in_specs=[blk, blk, dblk, dblk, blk, wblk, wblk, bblk],
        out_specs=blk,
        out_shape=jax.ShapeDtypeStruct((N_PAD, D), jnp.float32),
    )


_dense_relu = _make_dense(True)
_dense_lin = _make_dense(False)


def kernel(x, adj_t, W1_l, b1_l, W1_r, W2_l, b2_l, W2_r):
    src = adj_t[0].astype(jnp.int32)
    dst = adj_t[1].astype(jnp.int32)
    pad = jnp.full((E_PAD - E,), N, jnp.int32)
    src2 = jnp.concatenate([src, pad]).reshape(ER2D, CH)
    dst2 = jnp.concatenate([dst, pad]).reshape(ER2D, CH)
    xp = jnp.concatenate([x, jnp.zeros((N_PAD - N, D), jnp.float32)])

    ones_in = jnp.ones((CH, DEGW), jnp.float32)
    zrow = jnp.zeros((ROWS_PT, D), jnp.float32)
    zrow16 = jnp.zeros((ROWS_PT, DEGW), jnp.float32)

    a0, a1, d0, d1 = _sc_agg(xp, src2, dst2, ones_in, zrow, zrow16)
    h = _dense_relu(a0, a1, d0, d1, xp, W1_l.T, W1_r.T, b1_l.reshape(1, D))
    b0, b1, _, _ = _sc_agg(h, src2, dst2, ones_in, zrow, zrow16)
    out = _dense_lin(b0, b1, d0, d1, h, W2_l.T, W2_r.T, b2_l.reshape(1, D))
    return out[:N]


# speedup vs baseline: 3.0436x; 3.0436x over previous
import functools

import jax
import jax.numpy as jnp
from jax import lax
from jax.experimental import pallas as pl
from jax.experimental.pallas import tpu as pltpu
from jax.experimental.pallas import tpu_sc as plsc

N = 10000
D = 128
H = D // 2
E = 320000

NC = 2
NS = 16
CH = 128
G = 80
EPT = G * CH
E_PAD = NC * NS * EPT
ER2D = E_PAD // CH
N_PAD = 10112
ROWS_PT = N_PAD // NS
DEGW = 16


def _sc_agg_body(h0, h1, src2, dst2, ones_in, zrow, zrow16,
                 a00, a01, a10, a11, deg0, deg1,
                 ones_v, srcs, dsts, rows, aggs, degs, sem):
    cid = lax.axis_index("c")
    sid = lax.axis_index("s")
    tid = cid * NS + sid
    zb = sid * ROWS_PT

    pltpu.sync_copy(ones_in, ones_v)
    rb = tid * G
    pltpu.sync_copy(src2.at[pl.ds(rb, G)], srcs)
    pltpu.sync_copy(dst2.at[pl.ds(rb, G)], dsts)

    for half, h_hbm, out0, out1 in ((0, h0, a00, a10), (1, h1, a01, a11)):
        pltpu.sync_copy(zrow, aggs.at[pl.ds(zb, ROWS_PT)])
        if half == 0:
            pltpu.sync_copy(zrow16, degs.at[pl.ds(zb, ROWS_PT)])
        plsc.subcore_barrier()

        def chunk(g, carry):
            sidx = srcs.at[g]
            didx = dsts.at[g]
            cp = pltpu.async_copy(h_hbm.at[sidx], rows, sem)
            if half == 0:
                pltpu.sync_copy(ones_v, degs.at[didx], add=True)
            cp.wait()
            pltpu.sync_copy(rows, aggs.at[didx], add=True)
            return carry

        lax.fori_loop(0, G, chunk, 0)
        plsc.subcore_barrier()

        @pl.when(cid == 0)
        def _():
            pltpu.sync_copy(aggs.at[pl.ds(zb, ROWS_PT)], out0.at[pl.ds(zb, ROWS_PT)])
            if half == 0:
                pltpu.sync_copy(degs.at[pl.ds(zb, ROWS_PT)], deg0.at[pl.ds(zb, ROWS_PT)])

        @pl.when(cid == 1)
        def _():
            pltpu.sync_copy(aggs.at[pl.ds(zb, ROWS_PT)], out1.at[pl.ds(zb, ROWS_PT)])
            if half == 0:
                pltpu.sync_copy(degs.at[pl.ds(zb, ROWS_PT)], deg1.at[pl.ds(zb, ROWS_PT)])


_sc_agg = pl.kernel(
    _sc_agg_body,
    out_type=[
        jax.ShapeDtypeStruct((N_PAD, H), jnp.float32),
        jax.ShapeDtypeStruct((N_PAD, H), jnp.float32),
        jax.ShapeDtypeStruct((N_PAD, H), jnp.float32),
        jax.ShapeDtypeStruct((N_PAD, H), jnp.float32),
        jax.ShapeDtypeStruct((N_PAD, DEGW), jnp.float32),
        jax.ShapeDtypeStruct((N_PAD, DEGW), jnp.float32),
    ],
    mesh=plsc.VectorSubcoreMesh(core_axis_name="c", subcore_axis_name="s"),
    compiler_params=pltpu.CompilerParams(use_tc_tiling_on_sc=False),
    scratch_types=[
        pltpu.VMEM((CH, DEGW), jnp.float32),
        pltpu.VMEM((G, CH), jnp.int32),
        pltpu.VMEM((G, CH), jnp.int32),
        pltpu.VMEM((CH, H), jnp.float32),
        pltpu.VMEM_SHARED((N_PAD, H), jnp.float32),
        pltpu.VMEM_SHARED((N_PAD, DEGW), jnp.float32),
        pltpu.SemaphoreType.DMA,
    ],
)


def _dense_body(a00, a01, a10, a11, d0, d1, x0, x1, wl, wr, b, o0, o1, *, relu):
    deg = jnp.maximum(d0[:, 0:1] + d1[:, 0:1], 1.0)
    mean = jnp.concatenate([a00[...] + a10[...], a01[...] + a11[...]], axis=1) / deg
    xf = jnp.concatenate([x0[...], x1[...]], axis=1)
    r = (jnp.dot(mean, wl[...], preferred_element_type=jnp.float32)
         + jnp.dot(xf, wr[...], preferred_element_type=jnp.float32)
         + b[...])
    if relu:
        r = jnp.maximum(r, 0.0)
    o0[...] = r[:, :H]
    o1[...] = r[:, H:]


def _make_dense(relu):
    hblk = pl.BlockSpec((CH, H), lambda i: (i, 0))
    dblk = pl.BlockSpec((CH, DEGW), lambda i: (i, 0))
    wblk = pl.BlockSpec((D, D), lambda i: (0, 0))
    bblk = pl.BlockSpec((1, D), lambda i: (0, 0))
    return pl.pallas_call(
        functools.partial(_dense_body, relu=relu),
        grid=(N_PAD // CH,),
        in_specs=[hblk, hblk, hblk, hblk, dblk, dblk, hblk, hblk, wblk, wblk, bblk],
        out_specs=[hblk, hblk],
        out_shape=[
            jax.ShapeDtypeStruct((N_PAD, H), jnp.float32),
            jax.ShapeDtypeStruct((N_PAD, H), jnp.float32),
        ],
    )


_dense_relu = _make_dense(True)
_dense_lin = _make_dense(False)


def kernel(x, adj_t, W1_l, b1_l, W1_r, W2_l, b2_l, W2_r):
    src = adj_t[0].astype(jnp.int32)
    dst = adj_t[1].astype(jnp.int32)
    pad = jnp.full((E_PAD - E,), N, jnp.int32)
    src2 = jnp.concatenate([src, pad]).reshape(ER2D, CH)
    dst2 = jnp.concatenate([dst, pad]).reshape(ER2D, CH)
    xp = jnp.concatenate([x, jnp.zeros((N_PAD - N, D), jnp.float32)])
    x0 = xp[:, :H]
    x1 = xp[:, H:]

    ones_in = jnp.ones((CH, DEGW), jnp.float32)
    zrow = jnp.zeros((ROWS_PT, H), jnp.float32)
    zrow16 = jnp.zeros((ROWS_PT, DEGW), jnp.float32)

    a00, a01, a10, a11, d0, d1 = _sc_agg(x0, x1, src2, dst2, ones_in, zrow, zrow16)
    h0, h1 = _dense_relu(a00, a01, a10, a11, d0, d1, x0, x1,
                         W1_l.T, W1_r.T, b1_l.reshape(1, D))
    b00, b01, b10, b11, _, _ = _sc_agg(h0, h1, src2, dst2, ones_in, zrow, zrow16)
    o0, o1 = _dense_lin(b00, b01, b10, b11, d0, d1, h0, h1,
                        W2_l.T, W2_r.T, b2_l.reshape(1, D))
    return jnp.concatenate([o0, o1], axis=1)[:N]


# speedup vs baseline: 5.6612x; 1.8600x over previous
import functools

import jax
import jax.numpy as jnp
from jax import lax
from jax.experimental import pallas as pl
from jax.experimental.pallas import tpu as pltpu
from jax.experimental.pallas import tpu_sc as plsc

N = 10000
D = 128
H = D // 2
E = 320000

NC = 2
NS = 16
CH = 128
G = 80
EPT = G * CH
E_PAD = NC * NS * EPT
ER2D = E_PAD // CH
N_PAD = 10112
ROWS_PT = N_PAD // NS
DEGW = 16


def _sc_agg_body(h0, h1, src2, dst2, ones_in, zrow, zrow16,
                 a00, a01, a10, a11, deg0, deg1,
                 ones_v, srcs, dsts, rows, aggs, degs, sem):
    cid = lax.axis_index("c")
    sid = lax.axis_index("s")
    tid = cid * NS + sid
    zb = sid * ROWS_PT

    pltpu.sync_copy(ones_in, ones_v)
    rb = tid * G
    pltpu.sync_copy(src2.at[pl.ds(rb, G)], srcs)
    pltpu.sync_copy(dst2.at[pl.ds(rb, G)], dsts)

    for half, h_hbm, out0, out1 in ((0, h0, a00, a10), (1, h1, a01, a11)):
        pltpu.sync_copy(zrow, aggs.at[pl.ds(zb, ROWS_PT)])
        if half == 0:
            pltpu.sync_copy(zrow16, degs.at[pl.ds(zb, ROWS_PT)])
        plsc.subcore_barrier()

        def chunk(g, carry):
            sidx = srcs.at[g]
            didx = dsts.at[g]
            cp = pltpu.async_copy(h_hbm.at[sidx], rows, sem)
            if half == 0:
                pltpu.sync_copy(ones_v, degs.at[didx], add=True)
            cp.wait()
            pltpu.sync_copy(rows, aggs.at[didx], add=True)
            return carry

        lax.fori_loop(0, G, chunk, 0)
        plsc.subcore_barrier()

        @pl.when(cid == 0)
        def _():
            pltpu.sync_copy(aggs.at[pl.ds(zb, ROWS_PT)], out0.at[pl.ds(zb, ROWS_PT)])
            if half == 0:
                pltpu.sync_copy(degs.at[pl.ds(zb, ROWS_PT)], deg0.at[pl.ds(zb, ROWS_PT)])

        @pl.when(cid == 1)
        def _():
            pltpu.sync_copy(aggs.at[pl.ds(zb, ROWS_PT)], out1.at[pl.ds(zb, ROWS_PT)])
            if half == 0:
                pltpu.sync_copy(degs.at[pl.ds(zb, ROWS_PT)], deg1.at[pl.ds(zb, ROWS_PT)])


_sc_agg = pl.kernel(
    _sc_agg_body,
    out_type=[
        jax.ShapeDtypeStruct((N_PAD, H), jnp.float32),
        jax.ShapeDtypeStruct((N_PAD, H), jnp.float32),
        jax.ShapeDtypeStruct((N_PAD, H), jnp.float32),
        jax.ShapeDtypeStruct((N_PAD, H), jnp.float32),
        jax.ShapeDtypeStruct((N_PAD, DEGW), jnp.float32),
        jax.ShapeDtypeStruct((N_PAD, DEGW), jnp.float32),
    ],
    mesh=plsc.VectorSubcoreMesh(core_axis_name="c", subcore_axis_name="s"),
    compiler_params=pltpu.CompilerParams(use_tc_tiling_on_sc=False),
    scratch_types=[
        pltpu.VMEM((CH, DEGW), jnp.float32),
        pltpu.VMEM((G, CH), jnp.int32),
        pltpu.VMEM((G, CH), jnp.int32),
        pltpu.VMEM((CH, H), jnp.float32),
        pltpu.VMEM_SHARED((N_PAD, H), jnp.float32),
        pltpu.VMEM_SHARED((N_PAD, DEGW), jnp.float32),
        pltpu.SemaphoreType.DMA,
    ],
)


def _dense_body(a00, a01, a10, a11, d0, d1, x0, x1, wl, wr, b, o0, o1, *, relu):
    deg = jnp.maximum(d0[:, 0:1] + d1[:, 0:1], 1.0)
    mean = jnp.concatenate([a00[...] + a10[...], a01[...] + a11[...]], axis=1) / deg
    xf = jnp.concatenate([x0[...], x1[...]], axis=1)
    r = (jnp.dot(mean, wl[...], preferred_element_type=jnp.float32)
         + jnp.dot(xf, wr[...], preferred_element_type=jnp.float32)
         + b[...])
    if relu:
        r = jnp.maximum(r, 0.0)
    o0[...] = r[:, :H]
    o1[...] = r[:, H:]


def _make_dense(relu):
    hblk = pl.BlockSpec((CH, H), lambda i: (i, 0))
    dblk = pl.BlockSpec((CH, DEGW), lambda i: (i, 0))
    wblk = pl.BlockSpec((D, D), lambda i: (0, 0))
    bblk = pl.BlockSpec((1, D), lambda i: (0, 0))
    return pl.pallas_call(
        functools.partial(_dense_body, relu=relu),
        grid=(N_PAD // CH,),
        in_specs=[hblk, hblk, hblk, hblk, dblk, dblk, hblk, hblk, wblk, wblk, bblk],
        out_specs=[hblk, hblk],
        out_shape=[
            jax.ShapeDtypeStruct((N_PAD, H), jnp.float32),
            jax.ShapeDtypeStruct((N_PAD, H), jnp.float32),
        ],
    )


_dense_relu = _make_dense(True)
_dense_lin = _make_dense(False)


def kernel(x, adj_t, W1_l, b1_l, W1_r, W2_l, b2_l, W2_r):
    src = adj_t[0].astype(jnp.int32)
    dst = adj_t[1].astype(jnp.int32)
    pad = N + jnp.arange(E_PAD - E, dtype=jnp.int32) % (N_PAD - N)
    src2 = jnp.concatenate([src, pad]).reshape(ER2D, CH)
    dst2 = jnp.concatenate([dst, pad]).reshape(ER2D, CH)
    xp = jnp.concatenate([x, jnp.zeros((N_PAD - N, D), jnp.float32)])
    x0 = xp[:, :H]
    x1 = xp[:, H:]

    ones_in = jnp.ones((CH, DEGW), jnp.float32)
    zrow = jnp.zeros((ROWS_PT, H), jnp.float32)
    zrow16 = jnp.zeros((ROWS_PT, DEGW), jnp.float32)

    a00, a01, a10, a11, d0, d1 = _sc_agg(x0, x1, src2, dst2, ones_in, zrow, zrow16)
    h0, h1 = _dense_relu(a00, a01, a10, a11, d0, d1, x0, x1,
                         W1_l.T, W1_r.T, b1_l.reshape(1, D))
    b00, b01, b10, b11, _, _ = _sc_agg(h0, h1, src2, dst2, ones_in, zrow, zrow16)
    o0, o1 = _dense_lin(b00, b01, b10, b11, d0, d1, h0, h1,
                        W2_l.T, W2_r.T, b2_l.reshape(1, D))
    return jnp.concatenate([o0, o1], axis=1)[:N]


# speedup vs baseline: 7.9442x; 1.4033x over previous
import functools

import jax
import jax.numpy as jnp
from jax import lax
from jax.experimental import pallas as pl
from jax.experimental.pallas import tpu as pltpu
from jax.experimental.pallas import tpu_sc as plsc

N = 10000
D = 128
H = D // 2
E = 320000

NC = 2
NS = 16
CH = 128
G = 80
EPT = G * CH
E_PAD = NC * NS * EPT
ER2D = E_PAD // CH
N_PAD = 10112
ROWS_PT = N_PAD // NS
DEGW = 16


def _sc_agg_body(h0, h1, src2, dst2, ones_in, zrow, zrow16,
                 a00, a01, a10, a11, deg0, deg1,
                 ones_v, srcs, dsts, rows_a, rows_b, aggs, degs, sem_a, sem_b):
    cid = lax.axis_index("c")
    sid = lax.axis_index("s")
    tid = cid * NS + sid
    zb = sid * ROWS_PT

    pltpu.sync_copy(ones_in, ones_v)
    rb = tid * G
    pltpu.sync_copy(src2.at[pl.ds(rb, G)], srcs)
    pltpu.sync_copy(dst2.at[pl.ds(rb, G)], dsts)

    for half, h_hbm, out0, out1 in ((0, h0, a00, a10), (1, h1, a01, a11)):
        pltpu.sync_copy(zrow, aggs.at[pl.ds(zb, ROWS_PT)])
        if half == 0:
            pltpu.sync_copy(zrow16, degs.at[pl.ds(zb, ROWS_PT)])
        plsc.subcore_barrier()

        pltpu.async_copy(h_hbm.at[srcs.at[0]], rows_a, sem_a)

        def chunk2(i, carry):
            g0 = 2 * i
            g1 = g0 + 1
            g2 = lax.rem(g0 + 2, G)
            pltpu.async_copy(h_hbm.at[srcs.at[g1]], rows_b, sem_b)
            if half == 0:
                pltpu.sync_copy(ones_v, degs.at[dsts.at[g0]], add=True)
            pltpu.make_async_copy(h_hbm.at[srcs.at[g0]], rows_a, sem_a).wait()
            pltpu.sync_copy(rows_a, aggs.at[dsts.at[g0]], add=True)
            pltpu.async_copy(h_hbm.at[srcs.at[g2]], rows_a, sem_a)
            if half == 0:
                pltpu.sync_copy(ones_v, degs.at[dsts.at[g1]], add=True)
            pltpu.make_async_copy(h_hbm.at[srcs.at[g1]], rows_b, sem_b).wait()
            pltpu.sync_copy(rows_b, aggs.at[dsts.at[g1]], add=True)
            return carry

        lax.fori_loop(0, G // 2, chunk2, 0)
        pltpu.make_async_copy(h_hbm.at[srcs.at[0]], rows_a, sem_a).wait()
        plsc.subcore_barrier()

        @pl.when(cid == 0)
        def _():
            pltpu.sync_copy(aggs.at[pl.ds(zb, ROWS_PT)], out0.at[pl.ds(zb, ROWS_PT)])
            if half == 0:
                pltpu.sync_copy(degs.at[pl.ds(zb, ROWS_PT)], deg0.at[pl.ds(zb, ROWS_PT)])

        @pl.when(cid == 1)
        def _():
            pltpu.sync_copy(aggs.at[pl.ds(zb, ROWS_PT)], out1.at[pl.ds(zb, ROWS_PT)])
            if half == 0:
                pltpu.sync_copy(degs.at[pl.ds(zb, ROWS_PT)], deg1.at[pl.ds(zb, ROWS_PT)])


_sc_agg = pl.kernel(
    _sc_agg_body,
    out_type=[
        jax.ShapeDtypeStruct((N_PAD, H), jnp.float32),
        jax.ShapeDtypeStruct((N_PAD, H), jnp.float32),
        jax.ShapeDtypeStruct((N_PAD, H), jnp.float32),
        jax.ShapeDtypeStruct((N_PAD, H), jnp.float32),
        jax.ShapeDtypeStruct((N_PAD, DEGW), jnp.float32),
        jax.ShapeDtypeStruct((N_PAD, DEGW), jnp.float32),
    ],
    mesh=plsc.VectorSubcoreMesh(core_axis_name="c", subcore_axis_name="s"),
    compiler_params=pltpu.CompilerParams(use_tc_tiling_on_sc=False),
    scratch_types=[
        pltpu.VMEM((CH, DEGW), jnp.float32),
        pltpu.VMEM((G, CH), jnp.int32),
        pltpu.VMEM((G, CH), jnp.int32),
        pltpu.VMEM((CH, H), jnp.float32),
        pltpu.VMEM((CH, H), jnp.float32),
        pltpu.VMEM_SHARED((N_PAD, H), jnp.float32),
        pltpu.VMEM_SHARED((N_PAD, DEGW), jnp.float32),
        pltpu.SemaphoreType.DMA,
        pltpu.SemaphoreType.DMA,
    ],
)


def _dense_body(a00, a01, a10, a11, d0, d1, x0, x1, wl, wr, b, o0, o1, *, relu):
    deg = jnp.maximum(d0[:, 0:1] + d1[:, 0:1], 1.0)
    mean = jnp.concatenate([a00[...] + a10[...], a01[...] + a11[...]], axis=1) / deg
    xf = jnp.concatenate([x0[...], x1[...]], axis=1)
    r = (jnp.dot(mean, wl[...], preferred_element_type=jnp.float32)
         + jnp.dot(xf, wr[...], preferred_element_type=jnp.float32)
         + b[...])
    if relu:
        r = jnp.maximum(r, 0.0)
    o0[...] = r[:, :H]
    o1[...] = r[:, H:]


def _make_dense(relu):
    hblk = pl.BlockSpec((CH, H), lambda i: (i, 0))
    dblk = pl.BlockSpec((CH, DEGW), lambda i: (i, 0))
    wblk = pl.BlockSpec((D, D), lambda i: (0, 0))
    bblk = pl.BlockSpec((1, D), lambda i: (0, 0))
    return pl.pallas_call(
        functools.partial(_dense_body, relu=relu),
        grid=(N_PAD // CH,),
        in_specs=[hblk, hblk, hblk, hblk, dblk, dblk, hblk, hblk, wblk, wblk, bblk],
        out_specs=[hblk, hblk],
        out_shape=[
            jax.ShapeDtypeStruct((N_PAD, H), jnp.float32),
            jax.ShapeDtypeStruct((N_PAD, H), jnp.float32),
        ],
    )


_dense_relu = _make_dense(True)
_dense_lin = _make_dense(False)


def kernel(x, adj_t, W1_l, b1_l, W1_r, W2_l, b2_l, W2_r):
    src = adj_t[0].astype(jnp.int32)
    dst = adj_t[1].astype(jnp.int32)
    pad = N + jnp.arange(E_PAD - E, dtype=jnp.int32) % (N_PAD - N)
    src2 = jnp.concatenate([src, pad]).reshape(ER2D, CH)
    dst2 = jnp.concatenate([dst, pad]).reshape(ER2D, CH)
    xp = jnp.concatenate([x, jnp.zeros((N_PAD - N, D), jnp.float32)])
    x0 = xp[:, :H]
    x1 = xp[:, H:]

    ones_in = jnp.ones((CH, DEGW), jnp.float32)
    zrow = jnp.zeros((ROWS_PT, H), jnp.float32)
    zrow16 = jnp.zeros((ROWS_PT, DEGW), jnp.float32)

    a00, a01, a10, a11, d0, d1 = _sc_agg(x0, x1, src2, dst2, ones_in, zrow, zrow16)
    h0, h1 = _dense_relu(a00, a01, a10, a11, d0, d1, x0, x1,
                         W1_l.T, W1_r.T, b1_l.reshape(1, D))
    b00, b01, b10, b11, _, _ = _sc_agg(h0, h1, src2, dst2, ones_in, zrow, zrow16)
    o0, o1 = _dense_lin(b00, b01, b10, b11, d0, d1, h0, h1,
                        W2_l.T, W2_r.T, b2_l.reshape(1, D))
    return jnp.concatenate([o0, o1], axis=1)[:N]


# speedup vs baseline: 8.1692x; 1.0283x over previous
import functools

import jax
import jax.numpy as jnp
from jax import lax
from jax.experimental import pallas as pl
from jax.experimental.pallas import tpu as pltpu
from jax.experimental.pallas import tpu_sc as plsc

N = 10000
D = 128
H = D // 2
E = 320000

NC = 2
NS = 16
CH = 128
G = 80
EPT = G * CH
E_PAD = NC * NS * EPT
ER2D = E_PAD // CH
N_PAD = 10112
ROWS_PT = N_PAD // NS
DEGW = 16


def _sc_agg_body(with_deg, h0, h1, src2, dst2, *refs):
    if with_deg:
        (ones_in, zrow, zrow16, a00, a01, a10, a11, deg0, deg1,
         ones_v, srcs, dsts, rows_a, rows_b, aggs, degs, sem_a, sem_b) = refs
    else:
        (zrow, a00, a01, a10, a11,
         srcs, dsts, rows_a, rows_b, aggs, sem_a, sem_b) = refs
    cid = lax.axis_index("c")
    sid = lax.axis_index("s")
    tid = cid * NS + sid
    zb = sid * ROWS_PT

    if with_deg:
        pltpu.sync_copy(ones_in, ones_v)
    rb = tid * G
    pltpu.sync_copy(src2.at[pl.ds(rb, G)], srcs)
    pltpu.sync_copy(dst2.at[pl.ds(rb, G)], dsts)

    for half, h_hbm, out0, out1 in ((0, h0, a00, a10), (1, h1, a01, a11)):
        pltpu.sync_copy(zrow, aggs.at[pl.ds(zb, ROWS_PT)])
        if half == 0 and with_deg:
            pltpu.sync_copy(zrow16, degs.at[pl.ds(zb, ROWS_PT)])
        plsc.subcore_barrier()

        pltpu.async_copy(h_hbm.at[srcs.at[0]], rows_a, sem_a)

        def chunk2(i, carry):
            g0 = 2 * i
            g1 = g0 + 1
            g2 = lax.rem(g0 + 2, G)
            pltpu.async_copy(h_hbm.at[srcs.at[g1]], rows_b, sem_b)
            if half == 0 and with_deg:
                pltpu.sync_copy(ones_v, degs.at[dsts.at[g0]], add=True)
            pltpu.make_async_copy(h_hbm.at[srcs.at[g0]], rows_a, sem_a).wait()
            pltpu.sync_copy(rows_a, aggs.at[dsts.at[g0]], add=True)
            pltpu.async_copy(h_hbm.at[srcs.at[g2]], rows_a, sem_a)
            if half == 0 and with_deg:
                pltpu.sync_copy(ones_v, degs.at[dsts.at[g1]], add=True)
            pltpu.make_async_copy(h_hbm.at[srcs.at[g1]], rows_b, sem_b).wait()
            pltpu.sync_copy(rows_b, aggs.at[dsts.at[g1]], add=True)
            return carry

        lax.fori_loop(0, G // 2, chunk2, 0)
        pltpu.make_async_copy(h_hbm.at[srcs.at[0]], rows_a, sem_a).wait()
        plsc.subcore_barrier()

        @pl.when(cid == 0)
        def _():
            pltpu.sync_copy(aggs.at[pl.ds(zb, ROWS_PT)], out0.at[pl.ds(zb, ROWS_PT)])
            if half == 0 and with_deg:
                pltpu.sync_copy(degs.at[pl.ds(zb, ROWS_PT)], deg0.at[pl.ds(zb, ROWS_PT)])

        @pl.when(cid == 1)
        def _():
            pltpu.sync_copy(aggs.at[pl.ds(zb, ROWS_PT)], out1.at[pl.ds(zb, ROWS_PT)])
            if half == 0 and with_deg:
                pltpu.sync_copy(degs.at[pl.ds(zb, ROWS_PT)], deg1.at[pl.ds(zb, ROWS_PT)])


def _make_sc_agg(with_deg):
    agg_t = jax.ShapeDtypeStruct((N_PAD, H), jnp.float32)
    deg_t = jax.ShapeDtypeStruct((N_PAD, DEGW), jnp.float32)
    out_type = [agg_t] * 4 + ([deg_t] * 2 if with_deg else [])
    scratch = (
        ([pltpu.VMEM((CH, DEGW), jnp.float32)] if with_deg else [])
        + [
            pltpu.VMEM((G, CH), jnp.int32),
            pltpu.VMEM((G, CH), jnp.int32),
            pltpu.VMEM((CH, H), jnp.float32),
            pltpu.VMEM((CH, H), jnp.float32),
            pltpu.VMEM_SHARED((N_PAD, H), jnp.float32),
        ]
        + ([pltpu.VMEM_SHARED((N_PAD, DEGW), jnp.float32)] if with_deg else [])
        + [pltpu.SemaphoreType.DMA, pltpu.SemaphoreType.DMA]
    )
    return pl.kernel(
        functools.partial(_sc_agg_body, with_deg),
        out_type=out_type,
        mesh=plsc.VectorSubcoreMesh(core_axis_name="c", subcore_axis_name="s"),
        compiler_params=pltpu.CompilerParams(use_tc_tiling_on_sc=False),
        scratch_types=scratch,
    )


_sc_agg_deg = _make_sc_agg(True)
_sc_agg_nodeg = _make_sc_agg(False)


def _dense_body(a00, a01, a10, a11, d0, d1, x0, x1, wl, wr, b, *outs, relu):
    deg = jnp.maximum(d0[:, 0:1] + d1[:, 0:1], 1.0)
    mean = jnp.concatenate([a00[...] + a10[...], a01[...] + a11[...]], axis=1) / deg
    xf = jnp.concatenate([x0[...], x1[...]], axis=1)
    r = (jnp.dot(mean, wl[...], preferred_element_type=jnp.float32)
         + jnp.dot(xf, wr[...], preferred_element_type=jnp.float32)
         + b[...])
    if relu:
        r = jnp.maximum(r, 0.0)
    if len(outs) == 2:
        outs[0][...] = r[:, :H]
        outs[1][...] = r[:, H:]
    else:
        outs[0][...] = r


def _make_dense(relu, split_out):
    hblk = pl.BlockSpec((CH, H), lambda i: (i, 0))
    fblk = pl.BlockSpec((CH, D), lambda i: (i, 0))
    dblk = pl.BlockSpec((CH, DEGW), lambda i: (i, 0))
    wblk = pl.BlockSpec((D, D), lambda i: (0, 0))
    bblk = pl.BlockSpec((1, D), lambda i: (0, 0))
    half_t = jax.ShapeDtypeStruct((N, H), jnp.float32)
    full_t = jax.ShapeDtypeStruct((N, D), jnp.float32)
    return pl.pallas_call(
        functools.partial(_dense_body, relu=relu),
        grid=(N_PAD // CH,),
        in_specs=[hblk, hblk, hblk, hblk, dblk, dblk, hblk, hblk, wblk, wblk, bblk],
        out_specs=[hblk, hblk] if split_out else [fblk],
        out_shape=[half_t, half_t] if split_out else [full_t],
    )


_dense_relu = _make_dense(True, True)
_dense_lin = _make_dense(False, False)


def kernel(x, adj_t, W1_l, b1_l, W1_r, W2_l, b2_l, W2_r):
    src = adj_t[0].astype(jnp.int32)
    dst = adj_t[1].astype(jnp.int32)
    npad = E_PAD - E
    pad_src = jnp.arange(npad, dtype=jnp.int32) % N
    pad_dst = N + jnp.arange(npad, dtype=jnp.int32) % (N_PAD - N)
    src2 = jnp.concatenate([src, pad_src]).reshape(ER2D, CH)
    dst2 = jnp.concatenate([dst, pad_dst]).reshape(ER2D, CH)
    x0 = x[:, :H]
    x1 = x[:, H:]

    ones_in = jnp.ones((CH, DEGW), jnp.float32)
    zrow = jnp.zeros((ROWS_PT, H), jnp.float32)
    zrow16 = jnp.zeros((ROWS_PT, DEGW), jnp.float32)

    a00, a01, a10, a11, d0, d1 = _sc_agg_deg(x0, x1, src2, dst2,
                                             ones_in, zrow, zrow16)
    h0, h1 = _dense_relu(a00, a01, a10, a11, d0, d1, x0, x1,
                         W1_l.T, W1_r.T, b1_l.reshape(1, D))
    b00, b01, b10, b11 = _sc_agg_nodeg(h0, h1, src2, dst2, zrow)
    (out,) = _dense_lin(b00, b01, b10, b11, d0, d1, h0, h1,
                        W2_l.T, W2_r.T, b2_l.reshape(1, D))
    return out


# speedup vs baseline: 9.6171x; 1.1772x over previous
import functools

import jax
import jax.numpy as jnp
from jax import lax
from jax.experimental import pallas as pl
from jax.experimental.pallas import tpu as pltpu
from jax.experimental.pallas import tpu_sc as plsc

N = 10000
D = 128
H = D // 2
E = 320000

NC = 2
NS = 16
CH = 128
G = 80
EPT = G * CH
E_PAD = NC * NS * EPT
ER2D = E_PAD // CH
N_PAD = 10112
ROWS_PT = N_PAD // NS
DEGW = 16


def _sc_agg_body(with_deg, h0, h1, src2, dst2, *refs):
    if with_deg:
        (ones_in, zrow, zrow16, a00, a01, a10, a11, deg0, deg1,
         ones_v, srcs, dsts, rows_a, rows_b, aggs, degs, sem_a, sem_b) = refs
    else:
        (zrow, a00, a01, a10, a11,
         srcs, dsts, rows_a, rows_b, aggs, sem_a, sem_b) = refs
    cid = lax.axis_index("c")
    sid = lax.axis_index("s")
    tid = cid * NS + sid
    zb = sid * ROWS_PT

    if with_deg:
        pltpu.sync_copy(ones_in, ones_v)
    rb = tid * G
    pltpu.sync_copy(src2.at[pl.ds(rb, G)], srcs)
    pltpu.sync_copy(dst2.at[pl.ds(rb, G)], dsts)

    for half, h_hbm, out0, out1 in ((0, h0, a00, a10), (1, h1, a01, a11)):
        pltpu.sync_copy(zrow, aggs.at[pl.ds(zb, ROWS_PT)])
        if half == 0 and with_deg:
            pltpu.sync_copy(zrow16, degs.at[pl.ds(zb, ROWS_PT)])
        plsc.subcore_barrier()

        pltpu.async_copy(h_hbm.at[srcs.at[0]], rows_a, sem_a)

        def chunk2(i, carry):
            g0 = 2 * i
            g1 = g0 + 1
            g2 = lax.rem(g0 + 2, G)
            pltpu.async_copy(h_hbm.at[srcs.at[g1]], rows_b, sem_b)
            if half == 0 and with_deg:
                pltpu.sync_copy(ones_v, degs.at[dsts.at[g0]], add=True)
            pltpu.make_async_copy(h_hbm.at[srcs.at[g0]], rows_a, sem_a).wait()
            pltpu.sync_copy(rows_a, aggs.at[dsts.at[g0]], add=True)
            pltpu.async_copy(h_hbm.at[srcs.at[g2]], rows_a, sem_a)
            if half == 0 and with_deg:
                pltpu.sync_copy(ones_v, degs.at[dsts.at[g1]], add=True)
            pltpu.make_async_copy(h_hbm.at[srcs.at[g1]], rows_b, sem_b).wait()
            pltpu.sync_copy(rows_b, aggs.at[dsts.at[g1]], add=True)
            return carry

        lax.fori_loop(0, G // 2, chunk2, 0)
        pltpu.make_async_copy(h_hbm.at[srcs.at[0]], rows_a, sem_a).wait()
        plsc.subcore_barrier()

        @pl.when(cid == 0)
        def _():
            pltpu.sync_copy(aggs.at[pl.ds(zb, ROWS_PT)], out0.at[pl.ds(zb, ROWS_PT)])
            if half == 0 and with_deg:
                pltpu.sync_copy(degs.at[pl.ds(zb, ROWS_PT)], deg0.at[pl.ds(zb, ROWS_PT)])

        @pl.when(cid == 1)
        def _():
            pltpu.sync_copy(aggs.at[pl.ds(zb, ROWS_PT)], out1.at[pl.ds(zb, ROWS_PT)])
            if half == 0 and with_deg:
                pltpu.sync_copy(degs.at[pl.ds(zb, ROWS_PT)], deg1.at[pl.ds(zb, ROWS_PT)])


def _make_sc_agg(with_deg):
    agg_t = jax.ShapeDtypeStruct((N_PAD, H), jnp.float32)
    deg_t = jax.ShapeDtypeStruct((N_PAD, DEGW), jnp.float32)
    out_type = [agg_t] * 4 + ([deg_t] * 2 if with_deg else [])
    scratch = (
        ([pltpu.VMEM((CH, DEGW), jnp.float32)] if with_deg else [])
        + [
            pltpu.VMEM((G, CH), jnp.int32),
            pltpu.VMEM((G, CH), jnp.int32),
            pltpu.VMEM((CH, H), jnp.float32),
            pltpu.VMEM((CH, H), jnp.float32),
            pltpu.VMEM_SHARED((N_PAD, H), jnp.float32),
        ]
        + ([pltpu.VMEM_SHARED((N_PAD, DEGW), jnp.float32)] if with_deg else [])
        + [pltpu.SemaphoreType.DMA, pltpu.SemaphoreType.DMA]
    )
    return pl.kernel(
        functools.partial(_sc_agg_body, with_deg),
        out_type=out_type,
        mesh=plsc.VectorSubcoreMesh(core_axis_name="c", subcore_axis_name="s"),
        compiler_params=pltpu.CompilerParams(use_tc_tiling_on_sc=False),
        scratch_types=scratch,
    )


_sc_agg_deg = _make_sc_agg(True)
_sc_agg_nodeg = _make_sc_agg(False)


def _dense_body(a00, a01, a10, a11, d0, d1, x0, x1, wl, wr, b, *outs, relu):
    deg = jnp.maximum(d0[:, 0:1] + d1[:, 0:1], 1.0)
    mean = jnp.concatenate([a00[...] + a10[...], a01[...] + a11[...]], axis=1) / deg
    xf = jnp.concatenate([x0[...], x1[...]], axis=1)
    r = (jnp.dot(mean, wl[...], preferred_element_type=jnp.float32)
         + jnp.dot(xf, wr[...], preferred_element_type=jnp.float32)
         + b[...])
    if relu:
        r = jnp.maximum(r, 0.0)
    if len(outs) == 2:
        outs[0][...] = r[:, :H]
        outs[1][...] = r[:, H:]
    else:
        outs[0][...] = r


BM = 632


def _make_dense(relu, split_out):
    hblk = pl.BlockSpec((BM, H), lambda i: (i, 0))
    fblk = pl.BlockSpec((BM, D), lambda i: (i, 0))
    dblk = pl.BlockSpec((BM, DEGW), lambda i: (i, 0))
    wblk = pl.BlockSpec((D, D), lambda i: (0, 0))
    bblk = pl.BlockSpec((1, D), lambda i: (0, 0))
    half_t = jax.ShapeDtypeStruct((N, H), jnp.float32)
    full_t = jax.ShapeDtypeStruct((N, D), jnp.float32)
    return pl.pallas_call(
        functools.partial(_dense_body, relu=relu),
        grid=(N_PAD // BM,),
        in_specs=[hblk, hblk, hblk, hblk, dblk, dblk, hblk, hblk, wblk, wblk, bblk],
        out_specs=[hblk, hblk] if split_out else [fblk],
        out_shape=[half_t, half_t] if split_out else [full_t],
    )


_dense_relu = _make_dense(True, True)
_dense_lin = _make_dense(False, False)


def kernel(x, adj_t, W1_l, b1_l, W1_r, W2_l, b2_l, W2_r):
    src = adj_t[0].astype(jnp.int32)
    dst = adj_t[1].astype(jnp.int32)
    npad = E_PAD - E
    pad_src = jnp.arange(npad, dtype=jnp.int32) % N
    pad_dst = N + jnp.arange(npad, dtype=jnp.int32) % (N_PAD - N)
    src2 = jnp.concatenate([src, pad_src]).reshape(ER2D, CH)
    dst2 = jnp.concatenate([dst, pad_dst]).reshape(ER2D, CH)
    x0 = x[:, :H]
    x1 = x[:, H:]

    ones_in = jnp.ones((CH, DEGW), jnp.float32)
    zrow = jnp.zeros((ROWS_PT, H), jnp.float32)
    zrow16 = jnp.zeros((ROWS_PT, DEGW), jnp.float32)

    a00, a01, a10, a11, d0, d1 = _sc_agg_deg(x0, x1, src2, dst2,
                                             ones_in, zrow, zrow16)
    h0, h1 = _dense_relu(a00, a01, a10, a11, d0, d1, x0, x1,
                         W1_l.T, W1_r.T, b1_l.reshape(1, D))
    b00, b01, b10, b11 = _sc_agg_nodeg(h0, h1, src2, dst2, zrow)
    (out,) = _dense_lin(b00, b01, b10, b11, d0, d1, h0, h1,
                        W2_l.T, W2_r.T, b2_l.reshape(1, D))
    return out


# speedup vs baseline: 10.5375x; 1.0957x over previous
import functools

import jax
import jax.numpy as jnp
from jax import lax
from jax.experimental import pallas as pl
from jax.experimental.pallas import tpu as pltpu
from jax.experimental.pallas import tpu_sc as plsc

N = 10000
D = 128
H = D // 2
E = 320000

NC = 2
NS = 16
CH = 128
G = 80
EPT = G * CH
E_PAD = NC * NS * EPT
ER2D = E_PAD // CH
N_PAD = 10112
ROWS_PT = N_PAD // NS
DEGW = 16


def _sc_agg_body(with_deg, h0, h1, src2, dst2, *refs):
    if with_deg:
        (ones_in, zrow, zrow16, a00, a01, a10, a11, deg0, deg1,
         ones_v, srcs, dsts, b0, b1, b2, b3, aggs, degs,
         gsem, ssem, dsem) = refs
    else:
        (zrow, a00, a01, a10, a11,
         srcs, dsts, b0, b1, b2, b3, aggs, gsem, ssem, dsem) = refs
        ones_v = degs = None
    bufs = (b0, b1, b2, b3)
    cid = lax.axis_index("c")
    sid = lax.axis_index("s")
    tid = cid * NS + sid
    zb = sid * ROWS_PT

    if with_deg:
        pltpu.sync_copy(ones_in, ones_v)
    rb = tid * G
    pltpu.sync_copy(src2.at[pl.ds(rb, G)], srcs)
    pltpu.sync_copy(dst2.at[pl.ds(rb, G)], dsts)

    for half, h_hbm, out0, out1 in ((0, h0, a00, a10), (1, h1, a01, a11)):
        pltpu.sync_copy(zrow, aggs.at[pl.ds(zb, ROWS_PT)])
        if half == 0 and with_deg:
            pltpu.sync_copy(zrow16, degs.at[pl.ds(zb, ROWS_PT)])
        plsc.subcore_barrier()

        deg_here = half == 0 and with_deg

        def emit(g, buf_cur, buf_next, drain):
            pltpu.make_async_copy(h_hbm.at[srcs.at[g]], buf_cur, gsem).wait()
            pltpu.async_copy(buf_cur, aggs.at[dsts.at[g]], ssem, add=True)
            if deg_here:
                pltpu.async_copy(ones_v, degs.at[dsts.at[g]], dsem, add=True)
            if drain:
                pltpu.make_async_copy(buf_cur, aggs.at[dsts.at[g]], ssem).wait()
                if deg_here:
                    pltpu.make_async_copy(ones_v, degs.at[dsts.at[g]], dsem).wait()
            gn = lax.rem(g + 2, G)
            pltpu.async_copy(h_hbm.at[srcs.at[gn]], buf_next, gsem)

        pltpu.async_copy(h_hbm.at[srcs.at[0]], bufs[0], gsem)
        pltpu.async_copy(h_hbm.at[srcs.at[1]], bufs[1], gsem)
        for j in range(4):
            emit(j, bufs[j], bufs[(j + 2) % 4], drain=j >= 2)

        def block4(i, carry):
            base = 4 * i
            for j in range(4):
                emit(base + j, bufs[j], bufs[(j + 2) % 4], drain=True)
            return carry

        lax.fori_loop(1, G // 4, block4, 0)
        for j in range(2):
            pltpu.make_async_copy(bufs[j], aggs.at[dsts.at[0]], ssem).wait()
            if deg_here:
                pltpu.make_async_copy(ones_v, degs.at[dsts.at[0]], dsem).wait()
            pltpu.make_async_copy(h_hbm.at[srcs.at[0]], bufs[j], gsem).wait()
        plsc.subcore_barrier()

        @pl.when(cid == 0)
        def _():
            pltpu.sync_copy(aggs.at[pl.ds(zb, ROWS_PT)], out0.at[pl.ds(zb, ROWS_PT)])
            if half == 0 and with_deg:
                pltpu.sync_copy(degs.at[pl.ds(zb, ROWS_PT)], deg0.at[pl.ds(zb, ROWS_PT)])

        @pl.when(cid == 1)
        def _():
            pltpu.sync_copy(aggs.at[pl.ds(zb, ROWS_PT)], out1.at[pl.ds(zb, ROWS_PT)])
            if half == 0 and with_deg:
                pltpu.sync_copy(degs.at[pl.ds(zb, ROWS_PT)], deg1.at[pl.ds(zb, ROWS_PT)])


def _make_sc_agg(with_deg):
    agg_t = jax.ShapeDtypeStruct((N_PAD, H), jnp.float32)
    deg_t = jax.ShapeDtypeStruct((N_PAD, DEGW), jnp.float32)
    out_type = [agg_t] * 4 + ([deg_t] * 2 if with_deg else [])
    scratch = (
        ([pltpu.VMEM((CH, DEGW), jnp.float32)] if with_deg else [])
        + [
            pltpu.VMEM((G, CH), jnp.int32),
            pltpu.VMEM((G, CH), jnp.int32),
            pltpu.VMEM((CH, H), jnp.float32),
            pltpu.VMEM((CH, H), jnp.float32),
            pltpu.VMEM((CH, H), jnp.float32),
            pltpu.VMEM((CH, H), jnp.float32),
            pltpu.VMEM_SHARED((N_PAD, H), jnp.float32),
        ]
        + ([pltpu.VMEM_SHARED((N_PAD, DEGW), jnp.float32)] if with_deg else [])
        + [pltpu.SemaphoreType.DMA, pltpu.SemaphoreType.DMA,
           pltpu.SemaphoreType.DMA]
    )
    return pl.kernel(
        functools.partial(_sc_agg_body, with_deg),
        out_type=out_type,
        mesh=plsc.VectorSubcoreMesh(core_axis_name="c", subcore_axis_name="s"),
        compiler_params=pltpu.CompilerParams(use_tc_tiling_on_sc=False),
        scratch_types=scratch,
    )


_sc_agg_deg = _make_sc_agg(True)
_sc_agg_nodeg = _make_sc_agg(False)


def _dense_body(a00, a01, a10, a11, d0, d1, x0, x1, wl, wr, b, *outs, relu):
    deg = jnp.maximum(d0[:, 0:1] + d1[:, 0:1], 1.0)
    mean = jnp.concatenate([a00[...] + a10[...], a01[...] + a11[...]], axis=1) / deg
    xf = jnp.concatenate([x0[...], x1[...]], axis=1)
    r = (jnp.dot(mean, wl[...], preferred_element_type=jnp.float32)
         + jnp.dot(xf, wr[...], preferred_element_type=jnp.float32)
         + b[...])
    if relu:
        r = jnp.maximum(r, 0.0)
    if len(outs) == 2:
        outs[0][...] = r[:, :H]
        outs[1][...] = r[:, H:]
    else:
        outs[0][...] = r


BM = 632


def _make_dense(relu, split_out):
    hblk = pl.BlockSpec((BM, H), lambda i: (i, 0))
    fblk = pl.BlockSpec((BM, D), lambda i: (i, 0))
    dblk = pl.BlockSpec((BM, DEGW), lambda i: (i, 0))
    wblk = pl.BlockSpec((D, D), lambda i: (0, 0))
    bblk = pl.BlockSpec((1, D), lambda i: (0, 0))
    half_t = jax.ShapeDtypeStruct((N, H), jnp.float32)
    full_t = jax.ShapeDtypeStruct((N, D), jnp.float32)
    return pl.pallas_call(
        functools.partial(_dense_body, relu=relu),
        grid=(N_PAD // BM,),
        in_specs=[hblk, hblk, hblk, hblk, dblk, dblk, hblk, hblk, wblk, wblk, bblk],
        out_specs=[hblk, hblk] if split_out else [fblk],
        out_shape=[half_t, half_t] if split_out else [full_t],
    )


_dense_relu = _make_dense(True, True)
_dense_lin = _make_dense(False, False)


def kernel(x, adj_t, W1_l, b1_l, W1_r, W2_l, b2_l, W2_r):
    src = adj_t[0].astype(jnp.int32)
    dst = adj_t[1].astype(jnp.int32)
    npad = E_PAD - E
    pad_src = jnp.arange(npad, dtype=jnp.int32) % N
    pad_dst = N + jnp.arange(npad, dtype=jnp.int32) % (N_PAD - N)
    src2 = jnp.concatenate([src, pad_src]).reshape(ER2D, CH)
    dst2 = jnp.concatenate([dst, pad_dst]).reshape(ER2D, CH)
    x0 = x[:, :H]
    x1 = x[:, H:]

    ones_in = jnp.ones((CH, DEGW), jnp.float32)
    zrow = jnp.zeros((ROWS_PT, H), jnp.float32)
    zrow16 = jnp.zeros((ROWS_PT, DEGW), jnp.float32)

    a00, a01, a10, a11, d0, d1 = _sc_agg_deg(x0, x1, src2, dst2,
                                             ones_in, zrow, zrow16)
    h0, h1 = _dense_relu(a00, a01, a10, a11, d0, d1, x0, x1,
                         W1_l.T, W1_r.T, b1_l.reshape(1, D))
    b00, b01, b10, b11 = _sc_agg_nodeg(h0, h1, src2, dst2, zrow)
    (out,) = _dense_lin(b00, b01, b10, b11, d0, d1, h0, h1,
                        W2_l.T, W2_r.T, b2_l.reshape(1, D))
    return out


# speedup vs baseline: 11.3593x; 1.0780x over previous
import functools

import jax
import jax.numpy as jnp
from jax import lax
from jax.experimental import pallas as pl
from jax.experimental.pallas import tpu as pltpu
from jax.experimental.pallas import tpu_sc as plsc

N = 10000
D = 128
H = D // 2
E = 320000

NC = 2
NS = 16
CH = 128
G = 80
EPT = G * CH
E_PAD = NC * NS * EPT
ER2D = E_PAD // CH
N_PAD = 10112
ROWS_PT = N_PAD // NS
DEGW = 16


def _sc_agg_body(with_deg, hv, srcA, srcB, dst2, *refs):
    if with_deg:
        (ones_in, zrow, zrow16, a00, a01, a10, a11, deg0, deg1,
         ones_v, srcsA, srcsB, dsts, b0, b1, b2, b3, aggs, degs,
         gsem, ssem, dsem) = refs
    else:
        (zrow, a00, a01, a10, a11,
         srcsA, srcsB, dsts, b0, b1, b2, b3, aggs, gsem, ssem, dsem) = refs
        ones_v = degs = None
    bufs = (b0, b1, b2, b3)
    cid = lax.axis_index("c")
    sid = lax.axis_index("s")
    tid = cid * NS + sid
    zb = sid * ROWS_PT

    if with_deg:
        pltpu.sync_copy(ones_in, ones_v)
    rb = tid * G
    pltpu.sync_copy(srcA.at[pl.ds(rb, G)], srcsA)
    pltpu.sync_copy(srcB.at[pl.ds(rb, G)], srcsB)
    pltpu.sync_copy(dst2.at[pl.ds(rb, G)], dsts)

    for half, srcs, out0, out1 in ((0, srcsA, a00, a10), (1, srcsB, a01, a11)):
        h_hbm = hv
        pltpu.sync_copy(zrow, aggs.at[pl.ds(zb, ROWS_PT)])
        if half == 0 and with_deg:
            pltpu.sync_copy(zrow16, degs.at[pl.ds(zb, ROWS_PT)])
        plsc.subcore_barrier()

        deg_here = half == 0 and with_deg

        def emit(g, buf_cur, buf_next, drain):
            pltpu.make_async_copy(h_hbm.at[srcs.at[g]], buf_cur, gsem).wait()
            pltpu.async_copy(buf_cur, aggs.at[dsts.at[g]], ssem, add=True)
            if deg_here:
                pltpu.async_copy(ones_v, degs.at[dsts.at[g]], dsem, add=True)
            if drain:
                pltpu.make_async_copy(buf_cur, aggs.at[dsts.at[g]], ssem).wait()
                if deg_here:
                    pltpu.make_async_copy(ones_v, degs.at[dsts.at[g]], dsem).wait()
            gn = lax.rem(g + 2, G)
            pltpu.async_copy(h_hbm.at[srcs.at[gn]], buf_next, gsem)

        pltpu.async_copy(h_hbm.at[srcs.at[0]], bufs[0], gsem)
        pltpu.async_copy(h_hbm.at[srcs.at[1]], bufs[1], gsem)
        for j in range(4):
            emit(j, bufs[j], bufs[(j + 2) % 4], drain=j >= 2)

        def block4(i, carry):
            base = 4 * i
            for j in range(4):
                emit(base + j, bufs[j], bufs[(j + 2) % 4], drain=True)
            return carry

        lax.fori_loop(1, G // 4, block4, 0)
        for j in range(2):
            pltpu.make_async_copy(bufs[j], aggs.at[dsts.at[0]], ssem).wait()
            if deg_here:
                pltpu.make_async_copy(ones_v, degs.at[dsts.at[0]], dsem).wait()
            pltpu.make_async_copy(h_hbm.at[srcs.at[0]], bufs[j], gsem).wait()
        plsc.subcore_barrier()

        @pl.when(cid == 0)
        def _():
            pltpu.sync_copy(aggs.at[pl.ds(zb, ROWS_PT)], out0.at[pl.ds(zb, ROWS_PT)])
            if half == 0 and with_deg:
                pltpu.sync_copy(degs.at[pl.ds(zb, ROWS_PT)], deg0.at[pl.ds(zb, ROWS_PT)])

        @pl.when(cid == 1)
        def _():
            pltpu.sync_copy(aggs.at[pl.ds(zb, ROWS_PT)], out1.at[pl.ds(zb, ROWS_PT)])
            if half == 0 and with_deg:
                pltpu.sync_copy(degs.at[pl.ds(zb, ROWS_PT)], deg1.at[pl.ds(zb, ROWS_PT)])


def _make_sc_agg(with_deg):
    agg_t = jax.ShapeDtypeStruct((N_PAD, H), jnp.float32)
    deg_t = jax.ShapeDtypeStruct((N_PAD, DEGW), jnp.float32)
    out_type = [agg_t] * 4 + ([deg_t] * 2 if with_deg else [])
    scratch = (
        ([pltpu.VMEM((CH, DEGW), jnp.float32)] if with_deg else [])
        + [
            pltpu.VMEM((G, CH), jnp.int32),
            pltpu.VMEM((G, CH), jnp.int32),
            pltpu.VMEM((G, CH), jnp.int32),
            pltpu.VMEM((CH, H), jnp.float32),
            pltpu.VMEM((CH, H), jnp.float32),
            pltpu.VMEM((CH, H), jnp.float32),
            pltpu.VMEM((CH, H), jnp.float32),
            pltpu.VMEM_SHARED((N_PAD, H), jnp.float32),
        ]
        + ([pltpu.VMEM_SHARED((N_PAD, DEGW), jnp.float32)] if with_deg else [])
        + [pltpu.SemaphoreType.DMA, pltpu.SemaphoreType.DMA,
           pltpu.SemaphoreType.DMA]
    )
    return pl.kernel(
        functools.partial(_sc_agg_body, with_deg),
        out_type=out_type,
        mesh=plsc.VectorSubcoreMesh(core_axis_name="c", subcore_axis_name="s"),
        compiler_params=pltpu.CompilerParams(use_tc_tiling_on_sc=False),
        scratch_types=scratch,
    )


_sc_agg_deg = _make_sc_agg(True)
_sc_agg_nodeg = _make_sc_agg(False)


def _dense_body(a00, a01, a10, a11, d0, d1, x, wl, wr, b, o, *, relu):
    deg = jnp.maximum(d0[:, 0:1] + d1[:, 0:1], 1.0)
    mean = jnp.concatenate([a00[...] + a10[...], a01[...] + a11[...]], axis=1) / deg
    r = (jnp.dot(mean, wl[...], preferred_element_type=jnp.float32)
         + jnp.dot(x[...], wr[...], preferred_element_type=jnp.float32)
         + b[...])
    if relu:
        r = jnp.maximum(r, 0.0)
    o[...] = r


BM = 632


def _make_dense(relu):
    hblk = pl.BlockSpec((BM, H), lambda i: (i, 0))
    fblk = pl.BlockSpec((BM, D), lambda i: (i, 0))
    dblk = pl.BlockSpec((BM, DEGW), lambda i: (i, 0))
    wblk = pl.BlockSpec((D, D), lambda i: (0, 0))
    bblk = pl.BlockSpec((1, D), lambda i: (0, 0))
    return pl.pallas_call(
        functools.partial(_dense_body, relu=relu),
        grid=(N_PAD // BM,),
        in_specs=[hblk, hblk, hblk, hblk, dblk, dblk, fblk, wblk, wblk, bblk],
        out_specs=fblk,
        out_shape=jax.ShapeDtypeStruct((N, D), jnp.float32),
    )


_dense_relu = _make_dense(True)
_dense_lin = _make_dense(False)


def kernel(x, adj_t, W1_l, b1_l, W1_r, W2_l, b2_l, W2_r):
    src = adj_t[0].astype(jnp.int32)
    dst = adj_t[1].astype(jnp.int32)
    npad = E_PAD - E
    pad_src = jnp.arange(npad, dtype=jnp.int32) % N
    pad_dst = N + jnp.arange(npad, dtype=jnp.int32) % (N_PAD - N)
    srcp = jnp.concatenate([src, pad_src])
    srcA = (srcp * 2).reshape(ER2D, CH)
    srcB = (srcp * 2 + 1).reshape(ER2D, CH)
    dst2 = jnp.concatenate([dst, pad_dst]).reshape(ER2D, CH)
    xv = x.reshape(2 * N, H)

    ones_in = jnp.ones((CH, DEGW), jnp.float32)
    zrow = jnp.zeros((ROWS_PT, H), jnp.float32)
    zrow16 = jnp.zeros((ROWS_PT, DEGW), jnp.float32)

    a00, a01, a10, a11, d0, d1 = _sc_agg_deg(xv, srcA, srcB, dst2,
                                             ones_in, zrow, zrow16)
    h = _dense_relu(a00, a01, a10, a11, d0, d1, x,
                    W1_l.T, W1_r.T, b1_l.reshape(1, D))
    b00, b01, b10, b11 = _sc_agg_nodeg(h.reshape(2 * N, H), srcA, srcB, dst2,
                                       zrow)
    return _dense_lin(b00, b01, b10, b11, d0, d1, h,
                      W2_l.T, W2_r.T, b2_l.reshape(1, D))
